# baseline (speedup 1.0000x reference)
import functools

import jax
import jax.numpy as jnp
from jax import lax
from jax.experimental import pallas as pl
from jax.experimental.pallas import tpu as pltpu
from jax.experimental.pallas import tpu_sc as plsc

N_NODES = 10000
N_EDGES = 320000
D_FEAT = 128
D_EDGE = 16
OUT_FEATS = 128

NC = 2
NS = 16
L = 16
DH = D_FEAT // NC
HW = 8
ROWS_T = N_NODES // NS
SUB = 5
ROWS_B = ROWS_T // SUB
K = 125
ET = N_EDGES // NS
NCH = ET // K
RING = 16
NSC = NCH // RING
DEPTH = 3


def _rsqrt_newton(x):
    i = plsc.bitcast(x, jnp.int32)
    i = jnp.int32(0x5F3759DF) - lax.shift_right_logical(i, 1)
    y = plsc.bitcast(i, jnp.float32)
    for _ in range(3):
        y = y * (jnp.float32(1.5) - jnp.float32(0.5) * x * y * y)
    return y


def _sc_body(feat_h, sd_h, ef_h, z64_h, z8_h, z16_h, onesrc_h, onedst_h,
             agg_out, agge_out, indeg_out,
             table, agg, hist, agge,
             ring, rows0, rows1, rows2, efb0, efb1, onesb, onedb, hb, nb,
             gs0, gs1, gs2, ss0, ss1, ss2, es0, es1, fs0, fs1, dsem, zsem):
    c = lax.axis_index("c")
    s = lax.axis_index("s")
    rowsb = (rows0, rows1, rows2)
    efbb = (efb0, efb1)
    gsem = (gs0, gs1, gs2)
    ssem = (ss0, ss1, ss2)
    esem = (es0, es1)
    fsem = (fs0, fs1)

    r0 = s * ROWS_T
    zd = [
        pltpu.async_copy(z64_h, agg.at[pl.ds(r0, ROWS_T)], zsem),
        pltpu.async_copy(z8_h, hist.at[pl.ds(r0, ROWS_T)], zsem),
        pltpu.async_copy(z16_h, agge.at[pl.ds(r0, ROWS_T)], zsem),
        pltpu.async_copy(onesrc_h, onesb, zsem),
        pltpu.async_copy(onedst_h, onedb, zsem),
    ]
    for d in zd:
        d.wait()
    plsc.subcore_barrier()

    def dsup(g, carry):
        pltpu.sync_copy(sd_h.at[pl.ds(2 * (s * NCH + g * RING), 2 * RING)],
                        ring)
        descs = []
        for j in range(RING):
            descs.append(pltpu.async_copy(
                onesb, hist.at[ring.at[2 * j]], dsem, add=True))

        @pl.when(g // (NSC // 2) == c)
        def _():
            descs2 = []
            for j in range(RING):
                descs2.append(pltpu.async_copy(
                    onedb, hist.at[ring.at[2 * j + 1]], dsem, add=True))
            ld = [None] * RING
            ad = [None] * RING
            for j in range(2):
                e0 = s * ET + (g * RING + j) * K
                ld[j] = pltpu.async_copy(ef_h.at[pl.ds(e0, K)], efbb[j],
                                         esem[j])
            for j in range(RING):
                b = j % 2
                ld[j].wait()
                ad[j] = pltpu.async_copy(efbb[b], agge.at[ring.at[2 * j + 1]],
                                         fsem[b], add=True)
                if j + 2 < RING:
                    ad[j].wait()
                    e0 = s * ET + (g * RING + j + 2) * K
                    ld[j + 2] = pltpu.async_copy(ef_h.at[pl.ds(e0, K)],
                                                 efbb[b], esem[b])
            ad[RING - 2].wait()
            ad[RING - 1].wait()
            for d in descs2:
                d.wait()

        for d in descs:
            d.wait()
        return carry

    lax.fori_loop(0, NSC, dsup, 0)
    plsc.subcore_barrier()

    fbs = (rows0, rows1)
    fd = [None] * SUB
    fd[0] = pltpu.async_copy(
        feat_h.at[pl.ds(s * ROWS_T, ROWS_B), pl.ds(c * DH, DH)], fbs[0], gs0)
    for b in range(SUB):
        rb = s * ROWS_T + b * ROWS_B
        fb = fbs[b % 2]
        fd[b].wait()
        if b + 1 < SUB:
            fd[b + 1] = pltpu.async_copy(
                feat_h.at[pl.ds(rb + ROWS_B, ROWS_B), pl.ds(c * DH, DH)],
                fbs[(b + 1) % 2], gs0 if (b + 1) % 2 == 0 else gs1)
        pltpu.sync_copy(hist.at[pl.ds(rb, ROWS_B)], hb.at[pl.ds(0, ROWS_B)])
        iota = lax.iota(jnp.int32, L)
        zero16 = jnp.zeros((L,), jnp.int32)
        for i in range(8):
            cnt = plsc.load_gather(hb, [iota + i * L, zero16])
            y = _rsqrt_newton(jnp.maximum(cnt, jnp.float32(1.0)))
            nb[pl.ds(i * L, L)] = y

        def sbody(r, carry):
            rr = jnp.full((L,), r, jnp.int32)
            sv = plsc.load_gather(nb, [rr])
            for q in range(DH // L):
                v = fb[r, pl.ds(q * L, L)]
                fb[r, pl.ds(q * L, L)] = v * sv
            return carry

        lax.fori_loop(0, ROWS_B, sbody, 0)
        pltpu.sync_copy(fb, table.at[pl.ds(rb, ROWS_B)])
    plsc.subcore_barrier()

    def msup(g, carry):
        pltpu.sync_copy(sd_h.at[pl.ds(2 * (s * NCH + g * RING), 2 * RING)],
                        ring)

        gd = [None] * RING
        sd = [None] * RING
        for j in range(DEPTH):
            gd[j] = pltpu.async_copy(table.at[ring.at[2 * j]], rowsb[j],
                                     gsem[j])
        for j in range(RING):
            b = j % DEPTH
            gd[j].wait()
            sd[j] = pltpu.async_copy(rowsb[b], agg.at[ring.at[2 * j + 1]],
                                     ssem[b], add=True)
            if j + DEPTH < RING:
                sd[j].wait()
                gd[j + DEPTH] = pltpu.async_copy(
                    table.at[ring.at[2 * (j + DEPTH)]], rowsb[b], gsem[b])
        for j in range(RING - DEPTH, RING):
            sd[j].wait()
        return carry

    lax.fori_loop(0, NSC, msup, 0)
    plsc.subcore_barrier()

    pltpu.sync_copy(agg.at[pl.ds(r0, ROWS_T)],
                    agg_out.at[pl.ds(r0, ROWS_T), pl.ds(c * DH, DH)])
    pltpu.sync_copy(agge.at[pl.ds(r0, ROWS_T)],
                    agge_out.at[c, pl.ds(r0, ROWS_T)])
    pltpu.sync_copy(hist.at[pl.ds(r0, ROWS_T)],
                    indeg_out.at[c, pl.ds(r0, ROWS_T)])


_sc_agg = pl.kernel(
    _sc_body,
    out_type=(
        jax.ShapeDtypeStruct((N_NODES, D_FEAT), jnp.float32),
        jax.ShapeDtypeStruct((NC, N_NODES, D_EDGE), jnp.bfloat16),
        jax.ShapeDtypeStruct((NC, N_NODES, HW), jnp.float32),
    ),
    mesh=plsc.VectorSubcoreMesh(core_axis_name="c", subcore_axis_name="s",
                                num_cores=NC, num_subcores=NS),
    scratch_types=[
        pltpu.VMEM_SHARED((N_NODES, DH), jnp.float32),
        pltpu.VMEM_SHARED((N_NODES, DH), jnp.float32),
        pltpu.VMEM_SHARED((N_NODES, HW), jnp.float32),
        pltpu.VMEM_SHARED((N_NODES, D_EDGE), jnp.bfloat16),
        pltpu.VMEM((2 * RING, K), jnp.int32),
        pltpu.VMEM((K, DH), jnp.float32),
        pltpu.VMEM((K, DH), jnp.float32),
        pltpu.VMEM((K, DH), jnp.float32),
        pltpu.VMEM((K, D_EDGE), jnp.bfloat16),
        pltpu.VMEM((K, D_EDGE), jnp.bfloat16),
        pltpu.VMEM((K, HW), jnp.float32),
        pltpu.VMEM((K, HW), jnp.float32),
        pltpu.VMEM((128, HW), jnp.float32),
        pltpu.VMEM((128,), jnp.float32),
        pltpu.SemaphoreType.DMA,
        pltpu.SemaphoreType.DMA,
        pltpu.SemaphoreType.DMA,
        pltpu.SemaphoreType.DMA,
        pltpu.SemaphoreType.DMA,
        pltpu.SemaphoreType.DMA,
        pltpu.SemaphoreType.DMA,
        pltpu.SemaphoreType.DMA,
        pltpu.SemaphoreType.DMA,
        pltpu.SemaphoreType.DMA,
        pltpu.SemaphoreType.DMA,
        pltpu.SemaphoreType.DMA,
    ],
    compiler_params=pltpu.CompilerParams(use_tc_tiling_on_sc=False,
                                         needs_layout_passes=False,
                                         skip_device_barrier=True),
)


def _tc_body(x_ref, e_ref, d_ref, w1_ref, w2_ref, b_ref, o_ref):
    x = x_ref[...]
    e = (e_ref[0] + e_ref[1]).astype(jnp.float32)
    acc = jnp.dot(x, w1_ref[...], preferred_element_type=jnp.float32)
    acc = acc + jnp.dot(e, w2_ref[...], preferred_element_type=jnp.float32)
    d = (d_ref[0] + d_ref[1])[:, HW // 2:HW // 2 + 1]
    norm = lax.rsqrt(jnp.maximum(d, jnp.float32(1.0)))
    o_ref[...] = acc * norm + b_ref[...]


_MB = 1000


def _tc_mm(agg, agge, indeg, w1, w2, b2):
    return pl.pallas_call(
        _tc_body,
        out_shape=jax.ShapeDtypeStruct((N_NODES, OUT_FEATS), jnp.float32),
        grid=(N_NODES // _MB,),
        in_specs=[
            pl.BlockSpec((_MB, D_FEAT), lambda i: (i, 0)),
            pl.BlockSpec((NC, _MB, D_EDGE), lambda i: (0, i, 0)),
            pl.BlockSpec((NC, _MB, HW), lambda i: (0, i, 0)),
            pl.BlockSpec((D_FEAT, OUT_FEATS), lambda i: (0, 0)),
            pl.BlockSpec((D_EDGE, OUT_FEATS), lambda i: (0, 0)),
            pl.BlockSpec((1, OUT_FEATS), lambda i: (0, 0)),
        ],
        out_specs=pl.BlockSpec((_MB, OUT_FEATS), lambda i: (i, 0)),
    )(agg, agge, indeg, w1, w2, b2)


@jax.jit
def kernel(feat, edge_index, edge_feat, weight, bias):
    src2 = edge_index[0].reshape(N_EDGES // K, K)
    dst2 = edge_index[1].reshape(N_EDGES // K, K)
    sd = jnp.stack([src2, dst2], axis=1).reshape(2 * (N_EDGES // K), K)
    z64 = jnp.zeros((ROWS_T, DH), jnp.float32)
    z8 = jnp.zeros((ROWS_T, HW), jnp.float32)
    z16 = jnp.zeros((ROWS_T, D_EDGE), jnp.bfloat16)
    half1 = jnp.concatenate([jnp.ones((K, HW // 2), jnp.float32),
                             jnp.zeros((K, HW // 2), jnp.float32)], axis=1)
    half2 = jnp.concatenate([jnp.zeros((K, HW // 2), jnp.float32),
                             jnp.ones((K, HW // 2), jnp.float32)], axis=1)
    ef16 = edge_feat.astype(jnp.bfloat16)
    agg, agge, indeg = _sc_agg(feat, sd, ef16, z64, z8, z16,
                               half1, half2)
    return _tc_mm(agg, agge, indeg, weight[:D_FEAT], weight[D_FEAT:],
                  bias.reshape(1, OUT_FEATS))


# speedup vs baseline: 1.0687x; 1.0687x over previous
import functools

import jax
import jax.numpy as jnp
from jax import lax
from jax.experimental import pallas as pl
from jax.experimental.pallas import tpu as pltpu
from jax.experimental.pallas import tpu_sc as plsc

N_NODES = 10000
N_EDGES = 320000
D_FEAT = 128
D_EDGE = 16
OUT_FEATS = 128

NC = 2
NS = 16
L = 16
DH = D_FEAT // NC
HW = 8
ROWS_T = N_NODES // NS
SUB = 5
ROWS_B = ROWS_T // SUB
K = 125
ET = N_EDGES // NS
NCH = ET // K
RING = 16
NSC = NCH // RING
DEPTH = 2


def _rsqrt_newton(x):
    i = plsc.bitcast(x, jnp.int32)
    i = jnp.int32(0x5F3759DF) - lax.shift_right_logical(i, 1)
    y = plsc.bitcast(i, jnp.float32)
    for _ in range(3):
        y = y * (jnp.float32(1.5) - jnp.float32(0.5) * x * y * y)
    return y


def _sc_body(feat_h, sd_h, ef_h, z64_h, z8_h, z16_h, onesrc_h, onedst_h,
             agg_out, agge_out, indeg_out,
             table, agg, hist, agge,
             ring, ring2, rows0, rows1, efb0, efb1, onesb, onedb, hb, nb,
             gs0, gs1, rs0, ss0, ss1, rs1, es0, es1, fs0, fs1, dsem, zsem):
    c = lax.axis_index("c")
    s = lax.axis_index("s")
    rowsb = (rows0, rows1)
    efbb = (efb0, efb1)
    gsem = (gs0, gs1)
    ssem = (ss0, ss1)
    esem = (es0, es1)
    fsem = (fs0, fs1)

    r0 = s * ROWS_T
    zd = [
        pltpu.async_copy(z64_h, agg.at[pl.ds(r0, ROWS_T)], zsem),
        pltpu.async_copy(z8_h, hist.at[pl.ds(r0, ROWS_T)], zsem),
        pltpu.async_copy(z16_h, agge.at[pl.ds(r0, ROWS_T)], zsem),
        pltpu.async_copy(onesrc_h, onesb, zsem),
        pltpu.async_copy(onedst_h, onedb, zsem),
    ]
    for d in zd:
        d.wait()
    plsc.subcore_barrier()

    def _ring_src(g):
        return sd_h.at[pl.ds(2 * (s * NCH + g * RING), 2 * RING)]

    def _deg_pipe(g, rg):
        descs = []
        for j in range(RING):
            descs.append(pltpu.async_copy(
                onesb, hist.at[rg.at[2 * j]], dsem, add=True))

        @pl.when(g // (NSC // 2) == c)
        def _():
            descs2 = []
            for j in range(RING):
                descs2.append(pltpu.async_copy(
                    onedb, hist.at[rg.at[2 * j + 1]], dsem, add=True))
            ld = [None] * RING
            ad = [None] * RING
            for j in range(2):
                e0 = s * ET + (g * RING + j) * K
                ld[j] = pltpu.async_copy(ef_h.at[pl.ds(e0, K)], efbb[j],
                                         esem[j])
            for j in range(RING):
                b = j % 2
                ld[j].wait()
                ad[j] = pltpu.async_copy(efbb[b], agge.at[rg.at[2 * j + 1]],
                                         fsem[b], add=True)
                if j + 2 < RING:
                    ad[j].wait()
                    e0 = s * ET + (g * RING + j + 2) * K
                    ld[j + 2] = pltpu.async_copy(ef_h.at[pl.ds(e0, K)],
                                                 efbb[b], esem[b])
            ad[RING - 2].wait()
            ad[RING - 1].wait()
            for d in descs2:
                d.wait()

        for d in descs:
            d.wait()

    def dsup(h, carry):
        g0 = 2 * h
        pltpu.make_async_copy(_ring_src(g0), ring, rs0).wait()
        d1 = pltpu.async_copy(_ring_src(g0 + 1), ring2, rs1)
        _deg_pipe(g0, ring)
        d1.wait()

        @pl.when(h + 1 < NSC // 2)
        def _():
            pltpu.async_copy(_ring_src(g0 + 2), ring, rs0)

        _deg_pipe(g0 + 1, ring2)
        return carry

    pltpu.async_copy(_ring_src(0), ring, rs0)
    lax.fori_loop(0, NSC // 2, dsup, 0)
    plsc.subcore_barrier()

    fbs = (rows0, rows1)
    fd = [None] * SUB
    fd[0] = pltpu.async_copy(
        feat_h.at[pl.ds(s * ROWS_T, ROWS_B), pl.ds(c * DH, DH)], fbs[0], gs0)
    for b in range(SUB):
        rb = s * ROWS_T + b * ROWS_B
        fb = fbs[b % 2]
        fd[b].wait()
        if b + 1 < SUB:
            fd[b + 1] = pltpu.async_copy(
                feat_h.at[pl.ds(rb + ROWS_B, ROWS_B), pl.ds(c * DH, DH)],
                fbs[(b + 1) % 2], gs0 if (b + 1) % 2 == 0 else gs1)
        pltpu.sync_copy(hist.at[pl.ds(rb, ROWS_B)], hb.at[pl.ds(0, ROWS_B)])
        iota = lax.iota(jnp.int32, L)
        zero16 = jnp.zeros((L,), jnp.int32)
        for i in range(8):
            cnt = plsc.load_gather(hb, [iota + i * L, zero16])
            y = _rsqrt_newton(jnp.maximum(cnt, jnp.float32(1.0)))
            nb[pl.ds(i * L, L)] = y

        def sbody(r, carry):
            rr = jnp.full((L,), r, jnp.int32)
            sv = plsc.load_gather(nb, [rr])
            for q in range(DH // L):
                v = fb[r, pl.ds(q * L, L)]
                fb[r, pl.ds(q * L, L)] = v * sv
            return carry

        lax.fori_loop(0, ROWS_B, sbody, 0)
        pltpu.sync_copy(fb, table.at[pl.ds(rb, ROWS_B)])
    plsc.subcore_barrier()

    def _edge_pipe(rg):
        gd = [None] * RING
        sd = [None] * RING
        for j in range(DEPTH):
            gd[j] = pltpu.async_copy(table.at[rg.at[2 * j]], rowsb[j],
                                     gsem[j])
        for j in range(RING):
            b = j % DEPTH
            gd[j].wait()
            sd[j] = pltpu.async_copy(rowsb[b], agg.at[rg.at[2 * j + 1]],
                                     ssem[b], add=True)
            if j + DEPTH < RING:
                sd[j].wait()
                gd[j + DEPTH] = pltpu.async_copy(
                    table.at[rg.at[2 * (j + DEPTH)]], rowsb[b], gsem[b])
        for j in range(RING - DEPTH, RING):
            sd[j].wait()

    def msup(h, carry):
        g0 = 2 * h
        pltpu.make_async_copy(_ring_src(g0), ring, rs0).wait()
        d1 = pltpu.async_copy(_ring_src(g0 + 1), ring2, rs1)
        _edge_pipe(ring)
        d1.wait()

        @pl.when(h + 1 < NSC // 2)
        def _():
            pltpu.async_copy(_ring_src(g0 + 2), ring, rs0)

        _edge_pipe(ring2)
        return carry

    pltpu.async_copy(_ring_src(0), ring, rs0)
    lax.fori_loop(0, NSC // 2, msup, 0)
    plsc.subcore_barrier()

    pltpu.sync_copy(agg.at[pl.ds(r0, ROWS_T)],
                    agg_out.at[pl.ds(r0, ROWS_T), pl.ds(c * DH, DH)])
    pltpu.sync_copy(agge.at[pl.ds(r0, ROWS_T)],
                    agge_out.at[c, pl.ds(r0, ROWS_T)])
    pltpu.sync_copy(hist.at[pl.ds(r0, ROWS_T)],
                    indeg_out.at[c, pl.ds(r0, ROWS_T)])


_sc_agg = pl.kernel(
    _sc_body,
    out_type=(
        jax.ShapeDtypeStruct((N_NODES, D_FEAT), jnp.float32),
        jax.ShapeDtypeStruct((NC, N_NODES, D_EDGE), jnp.float32),
        jax.ShapeDtypeStruct((NC, N_NODES, HW), jnp.float32),
    ),
    mesh=plsc.VectorSubcoreMesh(core_axis_name="c", subcore_axis_name="s",
                                num_cores=NC, num_subcores=NS),
    scratch_types=[
        pltpu.VMEM_SHARED((N_NODES, DH), jnp.float32),
        pltpu.VMEM_SHARED((N_NODES, DH), jnp.float32),
        pltpu.VMEM_SHARED((N_NODES, HW), jnp.float32),
        pltpu.VMEM_SHARED((N_NODES, D_EDGE), jnp.float32),
        pltpu.VMEM((2 * RING, K), jnp.int32),
        pltpu.VMEM((2 * RING, K), jnp.int32),
        pltpu.VMEM((K, DH), jnp.float32),
        pltpu.VMEM((K, DH), jnp.float32),
        pltpu.VMEM((K, D_EDGE), jnp.float32),
        pltpu.VMEM((K, D_EDGE), jnp.float32),
        pltpu.VMEM((K, HW), jnp.float32),
        pltpu.VMEM((K, HW), jnp.float32),
        pltpu.VMEM((128, HW), jnp.float32),
        pltpu.VMEM((128,), jnp.float32),
        pltpu.SemaphoreType.DMA,
        pltpu.SemaphoreType.DMA,
        pltpu.SemaphoreType.DMA,
        pltpu.SemaphoreType.DMA,
        pltpu.SemaphoreType.DMA,
        pltpu.SemaphoreType.DMA,
        pltpu.SemaphoreType.DMA,
        pltpu.SemaphoreType.DMA,
        pltpu.SemaphoreType.DMA,
        pltpu.SemaphoreType.DMA,
        pltpu.SemaphoreType.DMA,
        pltpu.SemaphoreType.DMA,
    ],
    compiler_params=pltpu.CompilerParams(use_tc_tiling_on_sc=False,
                                         needs_layout_passes=False,
                                         skip_device_barrier=True),
)


def _tc_body(x_ref, e_ref, d_ref, w1_ref, w2_ref, b_ref, o_ref):
    x = x_ref[...]
    e = e_ref[0] + e_ref[1]
    acc = jnp.dot(x, w1_ref[...], preferred_element_type=jnp.float32)
    acc = acc + jnp.dot(e, w2_ref[...], preferred_element_type=jnp.float32)
    d = (d_ref[0] + d_ref[1])[:, HW // 2:HW // 2 + 1]
    norm = lax.rsqrt(jnp.maximum(d, jnp.float32(1.0)))
    o_ref[...] = acc * norm + b_ref[...]


_MB = 1000


def _tc_mm(agg, agge, indeg, w1, w2, b2):
    return pl.pallas_call(
        _tc_body,
        out_shape=jax.ShapeDtypeStruct((N_NODES, OUT_FEATS), jnp.float32),
        grid=(N_NODES // _MB,),
        in_specs=[
            pl.BlockSpec((_MB, D_FEAT), lambda i: (i, 0)),
            pl.BlockSpec((NC, _MB, D_EDGE), lambda i: (0, i, 0)),
            pl.BlockSpec((NC, _MB, HW), lambda i: (0, i, 0)),
            pl.BlockSpec((D_FEAT, OUT_FEATS), lambda i: (0, 0)),
            pl.BlockSpec((D_EDGE, OUT_FEATS), lambda i: (0, 0)),
            pl.BlockSpec((1, OUT_FEATS), lambda i: (0, 0)),
        ],
        out_specs=pl.BlockSpec((_MB, OUT_FEATS), lambda i: (i, 0)),
    )(agg, agge, indeg, w1, w2, b2)


@jax.jit
def kernel(feat, edge_index, edge_feat, weight, bias):
    src2 = edge_index[0].reshape(N_EDGES // K, K)
    dst2 = edge_index[1].reshape(N_EDGES // K, K)
    sd = jnp.stack([src2, dst2], axis=1).reshape(2 * (N_EDGES // K), K)
    z64 = jnp.zeros((ROWS_T, DH), jnp.float32)
    z8 = jnp.zeros((ROWS_T, HW), jnp.float32)
    z16 = jnp.zeros((ROWS_T, D_EDGE), jnp.float32)
    half1 = jnp.concatenate([jnp.ones((K, HW // 2), jnp.float32),
                             jnp.zeros((K, HW // 2), jnp.float32)], axis=1)
    half2 = jnp.concatenate([jnp.zeros((K, HW // 2), jnp.float32),
                             jnp.ones((K, HW // 2), jnp.float32)], axis=1)
    agg, agge, indeg = _sc_agg(feat, sd, edge_feat, z64, z8, z16,
                               half1, half2)
    return _tc_mm(agg, agge, indeg, weight[:D_FEAT], weight[D_FEAT:],
                  bias.reshape(1, OUT_FEATS))


# speedup vs baseline: 1.0692x; 1.0004x over previous
import functools

import jax
import jax.numpy as jnp
from jax import lax
from jax.experimental import pallas as pl
from jax.experimental.pallas import tpu as pltpu
from jax.experimental.pallas import tpu_sc as plsc

N_NODES = 10000
N_EDGES = 320000
D_FEAT = 128
D_EDGE = 16
OUT_FEATS = 128

NC = 2
NS = 16
L = 16
DH = D_FEAT // NC
HW = 8
ROWS_T = N_NODES // NS
SUB = 5
ROWS_B = ROWS_T // SUB
K = 125
ET = N_EDGES // NS
NCH = ET // K
RING = 16
NSC = NCH // RING
DEPTH = 2


def _rsqrt_newton(x):
    i = plsc.bitcast(x, jnp.int32)
    i = jnp.int32(0x5F3759DF) - lax.shift_right_logical(i, 1)
    y = plsc.bitcast(i, jnp.float32)
    for _ in range(3):
        y = y * (jnp.float32(1.5) - jnp.float32(0.5) * x * y * y)
    return y


def _sc_body(feat_h, sd_h, ef_h, z64_h, z8_h, z16_h, onesrc_h, onedst_h,
             agg_out, agge_out, indeg_out,
             table, agg, hist, agge,
             ring, ring2, rows0, rows1, efb0, efb1, onesb, onedb, hb, nb,
             gs0, gs1, rs0, ss0, ss1, rs1, es0, es1, fs0, fs1, dsem, zsem):
    c = lax.axis_index("c")
    s = lax.axis_index("s")
    rowsb = (rows0, rows1)
    efbb = (efb0, efb1)
    gsem = (gs0, gs1)
    ssem = (ss0, ss1)
    esem = (es0, es1)
    fsem = (fs0, fs1)

    r0 = s * ROWS_T
    zd = [
        pltpu.async_copy(z64_h, agg.at[pl.ds(r0, ROWS_T)], zsem),
        pltpu.async_copy(z8_h, hist.at[pl.ds(r0, ROWS_T)], zsem),
        pltpu.async_copy(z16_h, agge.at[pl.ds(r0, ROWS_T)], zsem),
        pltpu.async_copy(onesrc_h, onesb, zsem),
        pltpu.async_copy(onedst_h, onedb, zsem),
    ]
    for d in zd:
        d.wait()
    plsc.subcore_barrier()

    def _ring_src(g):
        return sd_h.at[pl.ds(2 * (s * NCH + g * RING), 2 * RING)]

    def _deg_pipe(g, rg):
        descs = []
        for j in range(RING):
            descs.append(pltpu.async_copy(
                onesb, hist.at[rg.at[2 * j]], dsem, add=True))

        @pl.when(g // (NSC // 2) == c)
        def _():
            descs2 = []
            for j in range(RING):
                descs2.append(pltpu.async_copy(
                    onedb, hist.at[rg.at[2 * j + 1]], dsem, add=True))
            ld = [None] * RING
            ad = [None] * RING
            for j in range(2):
                e0 = s * ET + (g * RING + j) * K
                ld[j] = pltpu.async_copy(ef_h.at[pl.ds(e0, K)], efbb[j],
                                         esem[j])
            for j in range(RING):
                b = j % 2
                ld[j].wait()
                ad[j] = pltpu.async_copy(efbb[b], agge.at[rg.at[2 * j + 1]],
                                         fsem[b], add=True)
                if j + 2 < RING:
                    ad[j].wait()
                    e0 = s * ET + (g * RING + j + 2) * K
                    ld[j + 2] = pltpu.async_copy(ef_h.at[pl.ds(e0, K)],
                                                 efbb[b], esem[b])
            ad[RING - 2].wait()
            ad[RING - 1].wait()
            for d in descs2:
                d.wait()

        for d in descs:
            d.wait()

    def dsup(h, carry):
        g0 = 2 * h
        pltpu.make_async_copy(_ring_src(g0), ring, rs0).wait()
        d1 = pltpu.async_copy(_ring_src(g0 + 1), ring2, rs1)
        _deg_pipe(g0, ring)
        d1.wait()

        @pl.when(h + 1 < NSC // 2)
        def _():
            pltpu.async_copy(_ring_src(g0 + 2), ring, rs0)

        _deg_pipe(g0 + 1, ring2)
        return carry

    pltpu.async_copy(_ring_src(0), ring, rs0)
    lax.fori_loop(0, NSC // 2, dsup, 0)
    plsc.subcore_barrier()

    fbs = (rows0, rows1)
    fd = [None] * SUB
    fd[0] = pltpu.async_copy(
        feat_h.at[pl.ds(s * ROWS_T, ROWS_B), pl.ds(c * DH, DH)], fbs[0], gs0)
    for b in range(SUB):
        rb = s * ROWS_T + b * ROWS_B
        fb = fbs[b % 2]
        fd[b].wait()
        if b + 1 < SUB:
            fd[b + 1] = pltpu.async_copy(
                feat_h.at[pl.ds(rb + ROWS_B, ROWS_B), pl.ds(c * DH, DH)],
                fbs[(b + 1) % 2], gs0 if (b + 1) % 2 == 0 else gs1)
        pltpu.sync_copy(hist.at[pl.ds(rb, ROWS_B)], hb.at[pl.ds(0, ROWS_B)])
        iota = lax.iota(jnp.int32, L)
        zero16 = jnp.zeros((L,), jnp.int32)
        for i in range(8):
            cnt = plsc.load_gather(hb, [iota + i * L, zero16])
            y = _rsqrt_newton(jnp.maximum(cnt, jnp.float32(1.0)))
            nb[pl.ds(i * L, L)] = y

        def sbody(r5, carry):
            for u in range(5):
                r = r5 * 5 + u
                rr = jnp.full((L,), r, jnp.int32)
                sv = plsc.load_gather(nb, [rr])
                for q in range(DH // L):
                    v = fb[r, pl.ds(q * L, L)]
                    fb[r, pl.ds(q * L, L)] = v * sv
            return carry

        lax.fori_loop(0, ROWS_B // 5, sbody, 0)
        pltpu.sync_copy(fb, table.at[pl.ds(rb, ROWS_B)])
    plsc.subcore_barrier()

    def _edge_pipe(rg):
        gd = [None] * RING
        sd = [None] * RING
        for j in range(DEPTH):
            gd[j] = pltpu.async_copy(table.at[rg.at[2 * j]], rowsb[j],
                                     gsem[j])
        for j in range(RING):
            b = j % DEPTH
            gd[j].wait()
            sd[j] = pltpu.async_copy(rowsb[b], agg.at[rg.at[2 * j + 1]],
                                     ssem[b], add=True)
            if j + DEPTH < RING:
                sd[j].wait()
                gd[j + DEPTH] = pltpu.async_copy(
                    table.at[rg.at[2 * (j + DEPTH)]], rowsb[b], gsem[b])
        for j in range(RING - DEPTH, RING):
            sd[j].wait()

    def msup(h, carry):
        g0 = 2 * h
        pltpu.make_async_copy(_ring_src(g0), ring, rs0).wait()
        d1 = pltpu.async_copy(_ring_src(g0 + 1), ring2, rs1)
        _edge_pipe(ring)
        d1.wait()

        @pl.when(h + 1 < NSC // 2)
        def _():
            pltpu.async_copy(_ring_src(g0 + 2), ring, rs0)

        _edge_pipe(ring2)
        return carry

    pltpu.async_copy(_ring_src(0), ring, rs0)
    lax.fori_loop(0, NSC // 2, msup, 0)
    plsc.subcore_barrier()

    pltpu.sync_copy(agg.at[pl.ds(r0, ROWS_T)],
                    agg_out.at[pl.ds(r0, ROWS_T), pl.ds(c * DH, DH)])
    pltpu.sync_copy(agge.at[pl.ds(r0, ROWS_T)],
                    agge_out.at[c, pl.ds(r0, ROWS_T)])
    pltpu.sync_copy(hist.at[pl.ds(r0, ROWS_T)],
                    indeg_out.at[c, pl.ds(r0, ROWS_T)])


_sc_agg = pl.kernel(
    _sc_body,
    out_type=(
        jax.ShapeDtypeStruct((N_NODES, D_FEAT), jnp.float32),
        jax.ShapeDtypeStruct((NC, N_NODES, D_EDGE), jnp.float32),
        jax.ShapeDtypeStruct((NC, N_NODES, HW), jnp.float32),
    ),
    mesh=plsc.VectorSubcoreMesh(core_axis_name="c", subcore_axis_name="s",
                                num_cores=NC, num_subcores=NS),
    scratch_types=[
        pltpu.VMEM_SHARED((N_NODES, DH), jnp.float32),
        pltpu.VMEM_SHARED((N_NODES, DH), jnp.float32),
        pltpu.VMEM_SHARED((N_NODES, HW), jnp.float32),
        pltpu.VMEM_SHARED((N_NODES, D_EDGE), jnp.float32),
        pltpu.VMEM((2 * RING, K), jnp.int32),
        pltpu.VMEM((2 * RING, K), jnp.int32),
        pltpu.VMEM((K, DH), jnp.float32),
        pltpu.VMEM((K, DH), jnp.float32),
        pltpu.VMEM((K, D_EDGE), jnp.float32),
        pltpu.VMEM((K, D_EDGE), jnp.float32),
        pltpu.VMEM((K, HW), jnp.float32),
        pltpu.VMEM((K, HW), jnp.float32),
        pltpu.VMEM((128, HW), jnp.float32),
        pltpu.VMEM((128,), jnp.float32),
        pltpu.SemaphoreType.DMA,
        pltpu.SemaphoreType.DMA,
        pltpu.SemaphoreType.DMA,
        pltpu.SemaphoreType.DMA,
        pltpu.SemaphoreType.DMA,
        pltpu.SemaphoreType.DMA,
        pltpu.SemaphoreType.DMA,
        pltpu.SemaphoreType.DMA,
        pltpu.SemaphoreType.DMA,
        pltpu.SemaphoreType.DMA,
        pltpu.SemaphoreType.DMA,
        pltpu.SemaphoreType.DMA,
    ],
    compiler_params=pltpu.CompilerParams(use_tc_tiling_on_sc=False,
                                         needs_layout_passes=False,
                                         skip_device_barrier=True),
)


def _tc_body(x_ref, e_ref, d_ref, w1_ref, w2_ref, b_ref, o_ref):
    x = x_ref[...]
    e = e_ref[0] + e_ref[1]
    acc = jnp.dot(x, w1_ref[...], preferred_element_type=jnp.float32)
    acc = acc + jnp.dot(e, w2_ref[...], preferred_element_type=jnp.float32)
    d = (d_ref[0] + d_ref[1])[:, HW // 2:HW // 2 + 1]
    norm = lax.rsqrt(jnp.maximum(d, jnp.float32(1.0)))
    o_ref[...] = acc * norm + b_ref[...]


_MB = 1000


def _tc_mm(agg, agge, indeg, w1, w2, b2):
    return pl.pallas_call(
        _tc_body,
        out_shape=jax.ShapeDtypeStruct((N_NODES, OUT_FEATS), jnp.float32),
        grid=(N_NODES // _MB,),
        in_specs=[
            pl.BlockSpec((_MB, D_FEAT), lambda i: (i, 0)),
            pl.BlockSpec((NC, _MB, D_EDGE), lambda i: (0, i, 0)),
            pl.BlockSpec((NC, _MB, HW), lambda i: (0, i, 0)),
            pl.BlockSpec((D_FEAT, OUT_FEATS), lambda i: (0, 0)),
            pl.BlockSpec((D_EDGE, OUT_FEATS), lambda i: (0, 0)),
            pl.BlockSpec((1, OUT_FEATS), lambda i: (0, 0)),
        ],
        out_specs=pl.BlockSpec((_MB, OUT_FEATS), lambda i: (i, 0)),
    )(agg, agge, indeg, w1, w2, b2)


@jax.jit
def kernel(feat, edge_index, edge_feat, weight, bias):
    src2 = edge_index[0].reshape(N_EDGES // K, K)
    dst2 = edge_index[1].reshape(N_EDGES // K, K)
    sd = jnp.stack([src2, dst2], axis=1).reshape(2 * (N_EDGES // K), K)
    z64 = jnp.zeros((ROWS_T, DH), jnp.float32)
    z8 = jnp.zeros((ROWS_T, HW), jnp.float32)
    z16 = jnp.zeros((ROWS_T, D_EDGE), jnp.float32)
    half1 = jnp.concatenate([jnp.ones((K, HW // 2), jnp.float32),
                             jnp.zeros((K, HW // 2), jnp.float32)], axis=1)
    half2 = jnp.concatenate([jnp.zeros((K, HW // 2), jnp.float32),
                             jnp.ones((K, HW // 2), jnp.float32)], axis=1)
    agg, agge, indeg = _sc_agg(feat, sd, edge_feat, z64, z8, z16,
                               half1, half2)
    return _tc_mm(agg, agge, indeg, weight[:D_FEAT], weight[D_FEAT:],
                  bias.reshape(1, OUT_FEATS))
